# Initial kernel scaffold; baseline (speedup 1.0000x reference)
import jax
import jax.numpy as jnp
from jax.experimental import pallas as pl


def kernel(layer_activation, head_weight):
    raise NotImplementedError("write your pallas kernel here")


# speedup vs baseline: 120.0089x; 120.0089x over previous
import jax
import jax.numpy as jnp
import numpy as np
from jax.experimental import pallas as pl
from jax.experimental.pallas import tpu as pltpu

_ALPHA = 100.0
_MB = 10
_CLS = 1000
_CP = 1024
_RB = 512

_THRESH = np.log(np.arange(1, _MB) / (_MB - np.arange(1, _MB))).astype(np.float32)


def _nac_kernel(act_ref, w_ref, out_ref):
    i = pl.program_id(0)
    act = act_ref[...]
    w = w_ref[...]
    a16 = act.astype(jnp.bfloat16)
    logits = jnp.dot(a16, w, preferred_element_type=jnp.float32)

    lane = jax.lax.broadcasted_iota(jnp.int32, logits.shape, 1)
    valid = lane < _CLS
    masked = jnp.where(valid, logits, -jnp.inf)
    m = jnp.max(masked, axis=1, keepdims=True)
    e = jnp.where(valid, jnp.exp(logits - m), 0.0)
    denom = jnp.sum(e, axis=1, keepdims=True)
    pu = e / denom - jnp.where(valid, 1.0 / _CLS, 0.0)

    grad = jax.lax.dot_general(
        pu.astype(jnp.bfloat16), w, (((1,), (1,)), ((), ())),
        preferred_element_type=jnp.float32)

    z = act * grad * _ALPHA

    rows = []
    c_prev = jnp.full((1, z.shape[1]), float(_RB), jnp.float32)
    for k in range(_MB - 1):
        c_k = jnp.sum((z >= _THRESH[k]).astype(jnp.float32), axis=0, keepdims=True)
        rows.append(c_prev - c_k)
        c_prev = c_k
    rows.append(c_prev)
    contrib = jnp.concatenate(rows, axis=0)

    @pl.when(i == 0)
    def _init():
        out_ref[...] = jnp.zeros_like(out_ref)

    out_ref[...] += contrib


def kernel(layer_activation, head_weight):
    B, N = layer_activation.shape
    C = head_weight.shape[1]
    wp = jnp.pad(head_weight, ((0, 0), (0, _CP - C))).astype(jnp.bfloat16)
    hist_t = pl.pallas_call(
        _nac_kernel,
        grid=(B // _RB,),
        in_specs=[
            pl.BlockSpec((_RB, N), lambda i: (i, 0)),
            pl.BlockSpec((N, _CP), lambda i: (0, 0)),
        ],
        out_specs=pl.BlockSpec((_MB, N), lambda i: (0, 0)),
        out_shape=jax.ShapeDtypeStruct((_MB, N), jnp.float32),
        compiler_params=pltpu.CompilerParams(
            dimension_semantics=("arbitrary",),
        ),
    )(layer_activation, wp)
    return hist_t.T


# speedup vs baseline: 150.6975x; 1.2557x over previous
import jax
import jax.numpy as jnp
from jax.experimental import pallas as pl
from jax.experimental.pallas import tpu as pltpu

_ALPHA = 100.0
_MB = 10
_CLS = 1000
_CP = 1024
_RB = 512

_FMASK = 0x071C71C7


def _nac_kernel(act_ref, w_ref, out_ref, acc_ref):
    i = pl.program_id(0)
    nsteps = pl.num_programs(0)
    act = act_ref[...]
    w = w_ref[...]
    a16 = act.astype(jnp.bfloat16)
    logits = jnp.dot(a16, w, preferred_element_type=jnp.float32)

    lane = jax.lax.broadcasted_iota(jnp.int32, logits.shape, 1)
    valid = lane < _CLS
    masked = jnp.where(valid, logits, -1e30)
    m = jnp.max(masked, axis=1, keepdims=True)
    e = jnp.exp(masked - m)
    denom = jnp.sum(e, axis=1, keepdims=True)
    pu = e / denom - jnp.where(valid, 1.0 / _CLS, 0.0)

    grad = jax.lax.dot_general(
        pu.astype(jnp.bfloat16), w, (((1,), (1,)), ((), ())),
        preferred_element_type=jnp.float32)

    zh = act * grad * (_ALPHA * 0.5)
    t = jnp.tanh(zh)
    b = jnp.minimum(jnp.floor(t * 5.0 + 5.0).astype(jnp.int32), _MB - 1)
    v = jnp.left_shift(jnp.int32(1), b * 3)

    h = v[: _RB // 2] + v[_RB // 2 :]
    h = h[: _RB // 4] + h[_RB // 4 :]
    pe = h & _FMASK
    po = jnp.right_shift(h, 3) & _FMASK

    def _halve3(x):
        x = x[:64] + x[64:]
        x = x[:32] + x[32:]
        return x[:16] + x[16:]

    qe = _halve3(pe)
    qo = _halve3(po)

    rows = []
    for mbin in range(_MB):
        q = qe if mbin % 2 == 0 else qo
        field = jnp.right_shift(q, 6 * (mbin // 2)) & 63
        rows.append(jnp.sum(field, axis=0, keepdims=True).astype(jnp.float32))
    contrib = jnp.concatenate(rows, axis=0)

    @pl.when(i == 0)
    def _init():
        acc_ref[...] = jnp.zeros_like(acc_ref)

    acc_ref[...] += contrib

    @pl.when(i == nsteps - 1)
    def _fin():
        out_ref[...] = jnp.transpose(acc_ref[...], (1, 0))


def kernel(layer_activation, head_weight):
    B, N = layer_activation.shape
    C = head_weight.shape[1]
    wp = jnp.pad(head_weight, ((0, 0), (0, _CP - C))).astype(jnp.bfloat16)
    return pl.pallas_call(
        _nac_kernel,
        grid=(B // _RB,),
        in_specs=[
            pl.BlockSpec((_RB, N), lambda i: (i, 0)),
            pl.BlockSpec((N, _CP), lambda i: (0, 0)),
        ],
        out_specs=pl.BlockSpec((N, _MB), lambda i: (0, 0)),
        out_shape=jax.ShapeDtypeStruct((N, _MB), jnp.float32),
        scratch_shapes=[pltpu.VMEM((_MB, N), jnp.float32)],
        compiler_params=pltpu.CompilerParams(
            dimension_semantics=("arbitrary",),
        ),
    )(layer_activation, wp)


# speedup vs baseline: 181.5437x; 1.2047x over previous
import jax
import jax.numpy as jnp
from jax.experimental import pallas as pl
from jax.experimental.pallas import tpu as pltpu

_ALPHA = 100.0
_MB = 10
_RB = 512

_FMASK = 0x071C71C7


def _nac_kernel(act_ref, w_ref, out_ref, acc_ref, w16_ref):
    i = pl.program_id(0)
    nsteps = pl.num_programs(0)

    @pl.when(i == 0)
    def _cast_w():
        w16_ref[...] = w_ref[...].astype(jnp.bfloat16)

    act = act_ref[...]
    w16 = w16_ref[...]
    a16 = act.astype(jnp.bfloat16)
    logits = jnp.dot(a16, w16, preferred_element_type=jnp.float32)

    m = jnp.max(logits, axis=1, keepdims=True)
    e = jnp.exp(logits - m)
    denom = jnp.sum(e, axis=1, keepdims=True)
    c = logits.shape[1]
    pu50 = (e / denom - 1.0 / c) * (_ALPHA * 0.5)

    grad50 = jax.lax.dot_general(
        pu50.astype(jnp.bfloat16), w16, (((1,), (1,)), ((), ())),
        preferred_element_type=jnp.float32)

    zh = act * grad50
    t = jnp.tanh(zh)
    b = jnp.minimum(t * 5.0 + 5.0, 9.0).astype(jnp.int32)
    v = jnp.left_shift(jnp.int32(1), b * 3)

    h = v[: _RB // 2] + v[_RB // 2 :]
    h = h[: _RB // 4] + h[_RB // 4 :]
    pe = h & _FMASK
    po = jnp.right_shift(h, 3) & _FMASK

    def _halve3(x):
        x = x[:64] + x[64:]
        x = x[:32] + x[32:]
        return x[:16] + x[16:]

    qe = _halve3(pe)
    qo = _halve3(po)

    rows = []
    for mbin in range(_MB):
        q = qe if mbin % 2 == 0 else qo
        field = jnp.right_shift(q, 6 * (mbin // 2)) & 63
        rows.append(jnp.sum(field, axis=0, keepdims=True).astype(jnp.float32))
    contrib = jnp.concatenate(rows, axis=0)

    @pl.when(i == 0)
    def _init():
        acc_ref[...] = jnp.zeros_like(acc_ref)

    acc_ref[...] += contrib

    @pl.when(i == nsteps - 1)
    def _fin():
        out_ref[...] = jnp.transpose(acc_ref[...], (1, 0))


def kernel(layer_activation, head_weight):
    B, N = layer_activation.shape
    C = head_weight.shape[1]
    return pl.pallas_call(
        _nac_kernel,
        grid=(B // _RB,),
        in_specs=[
            pl.BlockSpec((_RB, N), lambda i: (i, 0)),
            pl.BlockSpec((N, C), lambda i: (0, 0)),
        ],
        out_specs=pl.BlockSpec((N, _MB), lambda i: (0, 0)),
        out_shape=jax.ShapeDtypeStruct((N, _MB), jnp.float32),
        scratch_shapes=[
            pltpu.VMEM((_MB, N), jnp.float32),
            pltpu.VMEM((N, C), jnp.bfloat16),
        ],
        compiler_params=pltpu.CompilerParams(
            dimension_semantics=("arbitrary",),
        ),
    )(layer_activation, head_weight)
